# baseline (speedup 1.0000x reference)
import functools

import jax
import jax.numpy as jnp
from jax import lax
from jax.experimental import pallas as pl
from jax.experimental.pallas import tpu as pltpu
from jax.experimental.pallas import tpu_sc as plsc

NC = 2
NS = 16
CH = 64
KD = 2


def _sc_accumulate(feat, idx0, idx1, counts, zrows, n_pad, cap_groups):
    rows_per_tile = n_pad // NS
    mesh = plsc.VectorSubcoreMesh(core_axis_name="c", subcore_axis_name="s")

    @functools.partial(
        pl.kernel,
        out_type=jax.ShapeDtypeStruct((NC, n_pad, 128), jnp.float32),
        mesh=mesh,
        scratch_types=[
            pltpu.VMEM((16,), jnp.int32),
            pltpu.VMEM((2, 2 * KD, CH), jnp.int32),
            pltpu.VMEM((2, KD, CH, 128), jnp.float32),
            pltpu.VMEM_SHARED((n_pad, 128), jnp.float32),
            pltpu.SemaphoreType.DMA,
            pltpu.SemaphoreType.DMA,
            pltpu.SemaphoreType.DMA,
        ],
        compiler_params=pltpu.CompilerParams(use_tc_tiling_on_sc=False, needs_layout_passes=False),
    )
    def k(f_hbm, idx0_hbm, idx1_hbm, cnt_hbm, zer_hbm, out_hbm,
          cntv, idx, rows, acc, gsem, ssem, isem):
        c = lax.axis_index("c")
        s = lax.axis_index("s")

        pltpu.sync_copy(zer_hbm, acc.at[pl.ds(s * rows_per_tile, rows_per_tile)])
        plsc.subcore_barrier()

        pltpu.sync_copy(cnt_hbm.at[c], cntv)
        lane = lax.broadcasted_iota(jnp.int32, (16,), 0)
        ng = jnp.max(jnp.where(lane == s, cntv[...], 0))
        base = s * ng

        def main(idx_hbm):
            def g_fire(m):
                for j in range(KD):
                    pltpu.async_copy(f_hbm.at[idx.at[m].at[j]],
                                     rows.at[m].at[j], gsem)

            def g_drain(m):
                for j in range(KD):
                    pltpu.make_async_copy(f_hbm.at[idx.at[m].at[j]],
                                          rows.at[m].at[j], gsem).wait()

            def s_fire(m):
                for j in range(KD):
                    pltpu.async_copy(rows.at[m].at[j],
                                     acc.at[idx.at[m].at[KD + j]], ssem,
                                     add=True)

            def s_drain(m):
                for j in range(KD):
                    pltpu.make_async_copy(rows.at[m].at[j],
                                          acc.at[idx.at[m].at[KD + j]],
                                          ssem).wait()

            def idx_load(m, grp):
                return pltpu.async_copy(idx_hbm.at[grp], idx.at[m], isem)

            idx_load(0, base).wait()
            g_fire(0)

            def phase(g, m):
                g_drain(m)

                @pl.when(g >= 1)
                def _():
                    s_drain(1 - m)

                @pl.when(g <= ng - 2)
                def _():
                    a = idx_load(1 - m, base + g + 1)
                    s_fire(m)
                    a.wait()
                    g_fire(1 - m)

                @pl.when(g == ng - 1)
                def _():
                    s_fire(m)

            def body(g2, _):
                phase(2 * g2, 0)
                phase(2 * g2 + 1, 1)
                return 0

            lax.fori_loop(0, ng // 2, body, 0)
            s_drain(1)

        @pl.when(ng > 0)
        def _():
            @pl.when(c == 0)
            def _():
                main(idx0_hbm)

            @pl.when(c == 1)
            def _():
                main(idx1_hbm)

        plsc.subcore_barrier()
        pltpu.sync_copy(
            acc.at[pl.ds(s * rows_per_tile, rows_per_tile)],
            out_hbm.at[c, pl.ds(s * rows_per_tile, rows_per_tile)],
        )

    return k(feat, idx0, idx1, counts, zrows)


def _tc_tail_body(acc_ref, lin_ref, mw_ref, mb_ref, g_ref, b_ref, out_ref,
                  *, n_nodes, bn_eps):
    n = n_nodes
    h = jnp.zeros((n, 128), dtype=jnp.float32)
    for k in range(2):
        h = h + jnp.dot(acc_ref[k, :n, :], lin_ref[k],
                        preferred_element_type=jnp.float32)
    z = jnp.dot(h, mw_ref[...].T, preferred_element_type=jnp.float32) + mb_ref[...]
    r = jnp.maximum(z, 0.0)
    mean = jnp.mean(r, axis=0, keepdims=True)
    var = jnp.mean((r - mean) * (r - mean), axis=0, keepdims=True)
    out_ref[...] = g_ref[...] * (r - mean) * lax.rsqrt(var + bn_eps) + b_ref[...]


def kernel(feature, sp_embeddings, edge_index, edge_order, linear, mlp_w,
           mlp_b, bn_gamma, bn_beta):
    n_nodes, in_feat = feature.shape
    e = edge_index.shape[1]
    assert in_feat == 128

    n_pad = NS * 8 * ((n_nodes + NS * 8) // (NS * 8))

    grp_edges = KD * CH
    gpt_max = 2 * ((e + 2 * NS * grp_edges - 1) // (2 * NS * grp_edges))
    cap_groups = NS * gpt_max
    cap_e = cap_groups * grp_edges

    src = edge_index[0].astype(jnp.int32)
    dst = edge_index[1].astype(jnp.int32)
    upd = jnp.stack([src, dst], axis=1)
    init = jnp.concatenate(
        [jnp.zeros((cap_e, 1), jnp.int32),
         jnp.full((cap_e, 1), n_nodes, jnp.int32)], axis=1)

    idx3d = []
    gpt = []
    for c in range(2):
        m = edge_order == c
        csum = jnp.cumsum(m.astype(jnp.int32))
        pos = jnp.where(m, csum - 1, cap_e)
        packed = init.at[pos].set(upd, mode="drop")
        srcc = packed[:, 0].reshape(cap_groups, KD, CH)
        dstc = packed[:, 1].reshape(cap_groups, KD, CH)
        idx3d.append(jnp.concatenate([srcc, dstc], axis=1))
        cnt = csum[-1]
        per2 = 2 * NS * grp_edges
        gpt.append(2 * ((cnt + per2 - 1) // per2))
    counts = jnp.broadcast_to(jnp.stack(gpt)[:, None], (2, 16)).astype(jnp.int32)
    zrows = jnp.zeros((n_pad // NS, 128), dtype=jnp.float32)

    acc = _sc_accumulate(feature, idx3d[0], idx3d[1], counts, zrows, n_pad,
                         cap_groups)

    tail = pl.pallas_call(
        functools.partial(_tc_tail_body, n_nodes=n_nodes, bn_eps=1e-5),
        out_shape=jax.ShapeDtypeStruct((n_nodes, 128), jnp.float32),
    )
    return tail(acc, linear, mlp_w, mlp_b.reshape(1, 128),
                bn_gamma.reshape(1, 128), bn_beta.reshape(1, 128))


# speedup vs baseline: 20.4964x; 20.4964x over previous
import functools

import jax
import jax.numpy as jnp
from jax import lax
from jax.experimental import pallas as pl
from jax.experimental.pallas import tpu as pltpu
from jax.experimental.pallas import tpu_sc as plsc

NC = 2
NS = 16
CH = 64
KDEPTH = 1


def _sc_accumulate(f0, f1, idx3d, zrows, two_n_pad, blocks_per_tile,
                   n_table):
    global _N_TABLE
    _N_TABLE = n_table
    rows_per_tile = two_n_pad // NS
    nb = blocks_per_tile
    mesh = plsc.VectorSubcoreMesh(core_axis_name="c", subcore_axis_name="s")

    @functools.partial(
        pl.kernel,
        out_type=jax.ShapeDtypeStruct((NC, two_n_pad, 64), jnp.float32),
        mesh=mesh,
        scratch_types=[
            pltpu.VMEM((2, 2 * KDEPTH, CH), jnp.int32),
            pltpu.VMEM((2, KDEPTH, CH, 64), jnp.float32),
            pltpu.VMEM_SHARED((two_n_pad, 64), jnp.float32),
            pltpu.VMEM_SHARED((_N_TABLE, 64), jnp.float32),
            pltpu.SemaphoreType.DMA,
            pltpu.SemaphoreType.DMA,
            pltpu.SemaphoreType.DMA,
        ],
        compiler_params=pltpu.CompilerParams(use_tc_tiling_on_sc=False),
    )
    def k(f0_hbm, f1_hbm, idx_hbm, zer_hbm, out_hbm,
          idx, rows, acc, table, gsem, ssem, isem):
        c = lax.axis_index("c")
        s = lax.axis_index("s")

        pltpu.sync_copy(zer_hbm, acc.at[pl.ds(s * rows_per_tile, rows_per_tile)])
        tslice = _N_TABLE // NS

        @pl.when(c == 0)
        def _():
            pltpu.sync_copy(f0_hbm.at[pl.ds(s * tslice, tslice)],
                            table.at[pl.ds(s * tslice, tslice)])

        @pl.when(c == 1)
        def _():
            pltpu.sync_copy(f1_hbm.at[pl.ds(s * tslice, tslice)],
                            table.at[pl.ds(s * tslice, tslice)])

        plsc.subcore_barrier()

        def main():
            def fire_gathers(m):
                for j in range(KDEPTH):
                    pltpu.async_copy(table.at[idx.at[m].at[j]],
                                     rows.at[m].at[j], gsem)

            def drain_gathers(m):
                for j in range(KDEPTH):
                    pltpu.make_async_copy(table.at[idx.at[m].at[j]],
                                          rows.at[m].at[j], gsem).wait()

            def fire_scatters(m):
                for j in range(KDEPTH):
                    pltpu.async_copy(rows.at[m].at[j],
                                     acc.at[idx.at[m].at[KDEPTH + j]], ssem,
                                     add=True)

            def drain_scatters(m):
                for j in range(KDEPTH):
                    pltpu.make_async_copy(rows.at[m].at[j],
                                          acc.at[idx.at[m].at[KDEPTH + j]],
                                          ssem).wait()

            def load_idx(m, blkid):
                return pltpu.async_copy(idx_hbm.at[blkid], idx.at[m], isem)

            load_idx(0, s * nb).wait()
            fire_gathers(0)

            def phase(i, m):
                drain_gathers(m)

                @pl.when(i >= 1)
                def _():
                    drain_scatters(1 - m)

                @pl.when(i <= nb - 2)
                def _():
                    a = load_idx(1 - m, s * nb + i + 1)
                    fire_scatters(m)
                    a.wait()
                    fire_gathers(1 - m)

                @pl.when(i == nb - 1)
                def _():
                    fire_scatters(m)

            def body(i2, _):
                phase(2 * i2, 0)
                phase(2 * i2 + 1, 1)
                return 0

            lax.fori_loop(0, nb // 2, body, 0)
            drain_scatters(1)

        main()

        plsc.subcore_barrier()
        pltpu.sync_copy(
            acc.at[pl.ds(s * rows_per_tile, rows_per_tile)],
            out_hbm.at[c, pl.ds(s * rows_per_tile, rows_per_tile)],
        )

    return k(f0, f1, idx3d, zrows)


def _tc_tail_body(acc_ref, lin_ref, mw_ref, mb_ref, g_ref, b_ref, out_ref,
                  *, n_nodes, bn_eps):
    n = n_nodes
    h = jnp.zeros((n, 128), dtype=jnp.float32)
    for k in range(2):
        for c in range(2):
            a = acc_ref[c, k * n:(k + 1) * n, :]
            w = lin_ref[k, c * 64:(c + 1) * 64, :]
            h = h + jnp.dot(a, w, preferred_element_type=jnp.float32)
    z = jnp.dot(h, mw_ref[...].T, preferred_element_type=jnp.float32) + mb_ref[...]
    r = jnp.maximum(z, 0.0)
    mean = jnp.mean(r, axis=0, keepdims=True)
    var = jnp.mean((r - mean) * (r - mean), axis=0, keepdims=True)
    out_ref[...] = g_ref[...] * (r - mean) * lax.rsqrt(var + bn_eps) + b_ref[...]


def kernel(feature, sp_embeddings, edge_index, edge_order, linear, mlp_w,
           mlp_b, bn_gamma, bn_beta):
    n_nodes, in_feat = feature.shape
    e = edge_index.shape[1]
    assert in_feat == 128

    two_n_pad = ((2 * n_nodes + NS * 8) // (NS * 8)) * (NS * 8)

    blk_edges = NS * CH * KDEPTH * 2
    e_pad = ((e + blk_edges - 1) // blk_edges) * blk_edges
    src = edge_index[0]
    cidx = edge_index[1] + edge_order * n_nodes
    if e_pad != e:
        pad = e_pad - e
        src = jnp.concatenate([src, jnp.zeros((pad,), jnp.int32)])
        cidx = jnp.concatenate(
            [cidx, jnp.full((pad,), 2 * n_nodes, jnp.int32)])
    blocks_per_tile = e_pad // (NS * CH * KDEPTH)
    nblk = NS * blocks_per_tile
    idx3d = jnp.concatenate(
        [src.reshape(nblk, KDEPTH, CH), cidx.reshape(nblk, KDEPTH, CH)],
        axis=1)
    f0 = feature[:, :64]
    f1 = feature[:, 64:]
    zrows = jnp.zeros((two_n_pad // NS, 64), dtype=jnp.float32)

    acc = _sc_accumulate(f0, f1, idx3d, zrows, two_n_pad, blocks_per_tile,
                         n_nodes)

    tail = pl.pallas_call(
        functools.partial(_tc_tail_body, n_nodes=n_nodes, bn_eps=1e-5),
        out_shape=jax.ShapeDtypeStruct((n_nodes, 128), jnp.float32),
    )
    return tail(acc, linear, mlp_w, mlp_b.reshape(1, 128),
                bn_gamma.reshape(1, 128), bn_beta.reshape(1, 128))


# speedup vs baseline: 22.0762x; 1.0771x over previous
import functools

import jax
import jax.numpy as jnp
from jax import lax
from jax.experimental import pallas as pl
from jax.experimental.pallas import tpu as pltpu
from jax.experimental.pallas import tpu_sc as plsc

NC = 2
NS = 16
CH = 80
KDEPTH = 1


def _sc_accumulate(f0, f1, idx3d, zrows, two_n_pad, blocks_per_tile,
                   n_table):
    global _N_TABLE
    _N_TABLE = n_table
    rows_per_tile = two_n_pad // NS
    nb = blocks_per_tile
    mesh = plsc.VectorSubcoreMesh(core_axis_name="c", subcore_axis_name="s")

    @functools.partial(
        pl.kernel,
        out_type=jax.ShapeDtypeStruct((NC, two_n_pad, 64), jnp.float32),
        mesh=mesh,
        scratch_types=[
            pltpu.VMEM((2, 2 * KDEPTH, CH), jnp.int32),
            pltpu.VMEM((2, KDEPTH, CH, 64), jnp.float32),
            pltpu.VMEM_SHARED((two_n_pad, 64), jnp.float32),
            pltpu.VMEM_SHARED((_N_TABLE, 64), jnp.float32),
            pltpu.SemaphoreType.DMA,
            pltpu.SemaphoreType.DMA,
            pltpu.SemaphoreType.DMA,
        ],
        compiler_params=pltpu.CompilerParams(use_tc_tiling_on_sc=False),
    )
    def k(f0_hbm, f1_hbm, idx_hbm, zer_hbm, out_hbm,
          idx, rows, acc, table, gsem, ssem, isem):
        c = lax.axis_index("c")
        s = lax.axis_index("s")

        pltpu.sync_copy(zer_hbm, acc.at[pl.ds(s * rows_per_tile, rows_per_tile)])
        tslice = _N_TABLE // NS

        @pl.when(c == 0)
        def _():
            pltpu.sync_copy(f0_hbm.at[pl.ds(s * tslice, tslice)],
                            table.at[pl.ds(s * tslice, tslice)])

        @pl.when(c == 1)
        def _():
            pltpu.sync_copy(f1_hbm.at[pl.ds(s * tslice, tslice)],
                            table.at[pl.ds(s * tslice, tslice)])

        plsc.subcore_barrier()

        def main():
            def fire_gathers(m):
                for j in range(KDEPTH):
                    pltpu.async_copy(table.at[idx.at[m].at[j]],
                                     rows.at[m].at[j], gsem)

            def drain_gathers(m):
                for j in range(KDEPTH):
                    pltpu.make_async_copy(table.at[idx.at[m].at[j]],
                                          rows.at[m].at[j], gsem).wait()

            def fire_scatters(m):
                for j in range(KDEPTH):
                    pltpu.async_copy(rows.at[m].at[j],
                                     acc.at[idx.at[m].at[KDEPTH + j]], ssem,
                                     add=True)

            def drain_scatters(m):
                for j in range(KDEPTH):
                    pltpu.make_async_copy(rows.at[m].at[j],
                                          acc.at[idx.at[m].at[KDEPTH + j]],
                                          ssem).wait()

            def load_idx(m, blkid):
                return pltpu.async_copy(idx_hbm.at[blkid], idx.at[m], isem)

            load_idx(0, s * nb).wait()
            fire_gathers(0)

            def phase(i, m):
                drain_gathers(m)

                @pl.when(i >= 1)
                def _():
                    drain_scatters(1 - m)

                @pl.when(i <= nb - 2)
                def _():
                    a = load_idx(1 - m, s * nb + i + 1)
                    fire_scatters(m)
                    a.wait()
                    fire_gathers(1 - m)

                @pl.when(i == nb - 1)
                def _():
                    fire_scatters(m)

            def body(i2, _):
                phase(2 * i2, 0)
                phase(2 * i2 + 1, 1)
                return 0

            lax.fori_loop(0, nb // 2, body, 0)
            drain_scatters(1)

        main()

        plsc.subcore_barrier()
        pltpu.sync_copy(
            acc.at[pl.ds(s * rows_per_tile, rows_per_tile)],
            out_hbm.at[c, pl.ds(s * rows_per_tile, rows_per_tile)],
        )

    return k(f0, f1, idx3d, zrows)


def _tc_tail_body(acc_ref, lin_ref, mw_ref, mb_ref, g_ref, b_ref, out_ref,
                  *, n_nodes, bn_eps):
    n = n_nodes
    h = jnp.zeros((n, 128), dtype=jnp.float32)
    for k in range(2):
        for c in range(2):
            a = acc_ref[c, k * n:(k + 1) * n, :]
            w = lin_ref[k, c * 64:(c + 1) * 64, :]
            h = h + jnp.dot(a, w, preferred_element_type=jnp.float32)
    z = jnp.dot(h, mw_ref[...].T, preferred_element_type=jnp.float32) + mb_ref[...]
    r = jnp.maximum(z, 0.0)
    mean = jnp.mean(r, axis=0, keepdims=True)
    var = jnp.mean((r - mean) * (r - mean), axis=0, keepdims=True)
    out_ref[...] = g_ref[...] * (r - mean) * lax.rsqrt(var + bn_eps) + b_ref[...]


def kernel(feature, sp_embeddings, edge_index, edge_order, linear, mlp_w,
           mlp_b, bn_gamma, bn_beta):
    n_nodes, in_feat = feature.shape
    e = edge_index.shape[1]
    assert in_feat == 128

    two_n_pad = ((2 * n_nodes + NS * 8) // (NS * 8)) * (NS * 8)

    blk_edges = NS * CH * KDEPTH * 2
    e_pad = ((e + blk_edges - 1) // blk_edges) * blk_edges
    src = edge_index[0]
    cidx = edge_index[1] + edge_order * n_nodes
    if e_pad != e:
        pad = e_pad - e
        src = jnp.concatenate([src, jnp.zeros((pad,), jnp.int32)])
        cidx = jnp.concatenate(
            [cidx, jnp.full((pad,), 2 * n_nodes, jnp.int32)])
    blocks_per_tile = e_pad // (NS * CH * KDEPTH)
    nblk = NS * blocks_per_tile
    idx3d = jnp.concatenate(
        [src.reshape(nblk, KDEPTH, CH), cidx.reshape(nblk, KDEPTH, CH)],
        axis=1)
    f0 = feature[:, :64]
    f1 = feature[:, 64:]
    zrows = jnp.zeros((two_n_pad // NS, 64), dtype=jnp.float32)

    acc = _sc_accumulate(f0, f1, idx3d, zrows, two_n_pad, blocks_per_tile,
                         n_nodes)

    tail = pl.pallas_call(
        functools.partial(_tc_tail_body, n_nodes=n_nodes, bn_eps=1e-5),
        out_shape=jax.ShapeDtypeStruct((n_nodes, 128), jnp.float32),
    )
    return tail(acc, linear, mlp_w, mlp_b.reshape(1, 128),
                bn_gamma.reshape(1, 128), bn_beta.reshape(1, 128))
